# baseline (speedup 1.0000x reference)
import functools

import jax
import jax.numpy as jnp
from jax import lax
from jax.experimental import pallas as pl
from jax.experimental.pallas import tpu as pltpu
from jax.experimental.pallas import tpu_sc as plsc

_N = 10000
_E = 160000
_D = 256
_G = 64
_T = 100

_NC = 2
_NS = 16
_NW = _NC * _NS

_EPW = 5120
_EPAD = _EPW * _NW
_EGRP = _EPW // 16

_RPT = 320
_NPAD = _NW * _RPT
_DUMMY = _RPT

_CHUNK = 48
_CAP = 5248
_TRASH = _CAP + 48
_CBUF = _CAP + 128

_RMAX = 160
_ESTRIDE = 384
_EFLAT = (_RMAX + 1) * _ESTRIDE
_PCAP = 8064
_PTRASH = _PCAP + 32

_BLK = 1024
_NBLK = _NPAD // _BLK


_SC_MESH = plsc.VectorSubcoreMesh(core_axis_name="c", subcore_axis_name="s",
                                  num_cores=_NC, num_subcores=_NS)


def _dense_body(h_ref, wm_ref, bm_ref, ws_ref, bs_ref, m_ref, s_ref, *,
                apply_relu):
    h = h_ref[...]
    if apply_relu:
        h = jnp.maximum(h, 0.0)
    i = pl.program_id(0)
    rowmask = (lax.broadcasted_iota(jnp.int32, (_BLK, 1), 0)
               + i * _BLK) < _N
    m = (jnp.dot(h, wm_ref[...], preferred_element_type=jnp.float32)
         + bm_ref[...])
    m_ref[...] = jnp.where(rowmask, m, 0.0)
    s_ref[...] = (jnp.dot(h, ws_ref[...], preferred_element_type=jnp.float32)
                  + bs_ref[...])


def _dense(h, wm, bm, ws, bs, apply_relu):
    return pl.pallas_call(
        functools.partial(_dense_body, apply_relu=apply_relu),
        grid=(_NBLK,),
        in_specs=[
            pl.BlockSpec((_BLK, _D), lambda i: (i, 0)),
            pl.BlockSpec((_D, _D), lambda i: (0, 0)),
            pl.BlockSpec((1, _D), lambda i: (0, 0)),
            pl.BlockSpec((_D, _D), lambda i: (0, 0)),
            pl.BlockSpec((1, _D), lambda i: (0, 0)),
        ],
        out_specs=[
            pl.BlockSpec((_BLK, _D), lambda i: (i, 0)),
            pl.BlockSpec((_BLK, _D), lambda i: (i, 0)),
        ],
        out_shape=[
            jax.ShapeDtypeStruct((_NPAD, _D), jnp.float32),
            jax.ShapeDtypeStruct((_NPAD, _D), jnp.float32),
        ],
    )(h, wm, bm.reshape(1, _D), ws, bs.reshape(1, _D))


def _bin_body(src_hbm, dst_hbm, bsrc_hbm, bloc_hbm, counts_hbm,
              src_v, dst_v, cand_v, counts_v, sem):
    cid = lax.axis_index("c")
    sid = lax.axis_index("s")
    w = cid * _NS + sid

    pltpu.sync_copy(src_hbm.at[pl.ds(w * _EPW, _EPW)], src_v)
    pltpu.sync_copy(dst_hbm.at[pl.ds(w * _EPW, _EPW)], dst_v)

    lanes = lax.iota(jnp.int32, 16)

    def _owner(o, carry):
        cnt_lo, cnt_hi = carry
        lo = o * _RPT
        soff = 0
        loff = _CBUF

        def _grp(g, ptr):
            s = src_v[pl.ds(g * 16, 16)]
            d = dst_v[pl.ds(g * 16, 16)]
            mask = (d >= lo) & (d < lo + _RPT)
            incl = plsc.cumsum(mask.astype(jnp.int32))
            pos = jnp.where(mask, ptr + incl - 1, _TRASH)
            plsc.store_scatter(cand_v, [soff + pos], s)
            plsc.store_scatter(cand_v, [loff + pos], d - lo)
            return ptr + incl[15]

        cnt = lax.fori_loop(0, _EGRP, _grp, jnp.int32(0), unroll=False)

        for k in range(_CHUNK // 16):
            pos = cnt + lanes + k * 16
            plsc.store_scatter(cand_v, [soff + pos],
                               jnp.zeros((16,), jnp.int32))
            plsc.store_scatter(cand_v, [loff + pos],
                               jnp.full((16,), _DUMMY, jnp.int32))

        pltpu.sync_copy(cand_v.at[pl.ds(0, _CAP)], bsrc_hbm.at[o, w])
        pltpu.sync_copy(cand_v.at[pl.ds(_CBUF, _CAP)], bloc_hbm.at[o, w])

        onehot = lanes == (o % 16)
        cnt_lo = jnp.where(onehot & (o < 16), cnt, cnt_lo)
        cnt_hi = jnp.where(onehot & (o >= 16), cnt, cnt_hi)
        return cnt_lo, cnt_hi

    cnt_lo, cnt_hi = lax.fori_loop(
        0, _NW, _owner,
        (jnp.zeros((16,), jnp.int32), jnp.zeros((16,), jnp.int32)),
        unroll=False)

    counts_v[pl.ds(0, 16)] = cnt_lo
    counts_v[pl.ds(16, 16)] = cnt_hi
    pltpu.sync_copy(counts_v, counts_hbm.at[w])


_bin_kernel = pl.kernel(
    _bin_body,
    out_type=[
        jax.ShapeDtypeStruct((_NW, _NW, _CAP), jnp.int32),
        jax.ShapeDtypeStruct((_NW, _NW, _CAP), jnp.int32),
        jax.ShapeDtypeStruct((_NW, _NW), jnp.int32),
    ],
    mesh=_SC_MESH,
    scratch_types=[
        pltpu.VMEM((_EPW,), jnp.int32),
        pltpu.VMEM((_EPW,), jnp.int32),
        pltpu.VMEM((2 * _CBUF,), jnp.int32),
        pltpu.VMEM((32,), jnp.int32),
        pltpu.SemaphoreType.DMA,
    ],
    compiler_params=pltpu.CompilerParams(needs_layout_passes=False),
)


def _ell_body(bsrc_hbm, bloc_hbm, countsT_hbm,
              plist_hbm, llist_hbm, rounds_hbm,
              ell_v, deg_v, cells_v, counts_v, rv_v,
              plist_v, llist_v, csem):
    cid = lax.axis_index("c")
    sid = lax.axis_index("s")
    w = cid * _NS + sid

    pltpu.sync_copy(countsT_hbm.at[w], counts_v)

    def _zero_deg(z, _):
        deg_v[pl.ds(z * 16, 16)] = jnp.zeros((16,), jnp.int32)
        return ()

    lax.fori_loop(0, _ESTRIDE // 16, _zero_deg, (), unroll=False)

    def _init_ell(z, _):
        ell_v[pl.ds(z * 16, 16)] = jnp.full((16,), _N, jnp.int32)
        return ()

    lax.fori_loop(0, _EFLAT // 16, _init_ell, (), unroll=False)

    pltpu.async_copy(bsrc_hbm.at[w, 0], cells_v.at[pl.ds(0, _CAP)], csem)
    pltpu.async_copy(bloc_hbm.at[w, 0], cells_v.at[pl.ds(_CAP, _CAP)], csem)

    def _writer(i, _):
        coff = (i % 2) * 2 * _CAP
        pltpu.make_async_copy(bsrc_hbm.at[w, i],
                              cells_v.at[pl.ds(coff, _CAP)], csem).wait()
        pltpu.make_async_copy(bloc_hbm.at[w, i],
                              cells_v.at[pl.ds(coff + _CAP, _CAP)],
                              csem).wait()
        cnt = plsc.load_gather(counts_v, [jnp.full((16,), i, jnp.int32)])[0]

        @pl.when(i < _NW - 1)
        def _():
            noff = 2 * _CAP - coff
            pltpu.async_copy(bsrc_hbm.at[w, i + 1],
                             cells_v.at[pl.ds(noff, _CAP)], csem)
            pltpu.async_copy(bloc_hbm.at[w, i + 1],
                             cells_v.at[pl.ds(noff + _CAP, _CAP)], csem)

        def _grp(g, _):
            srcv = cells_v[pl.ds(coff + g * 16, 16)]
            locv = cells_v[pl.ds(coff + _CAP + g * 16, 16)]
            ordinal, last = plsc.scan_count(locv)
            degv = plsc.load_gather(deg_v, [locv])
            r = degv + ordinal - 1
            rc = jnp.minimum(r, _RMAX)
            plsc.store_scatter(ell_v, [rc * _ESTRIDE + locv], srcv)
            plsc.store_scatter(deg_v, [locv], r + 1, mask=last)
            return ()

        ngrp = lax.div(cnt + 15, jnp.int32(16))
        lax.fori_loop(0, ngrp, _grp, (), unroll=False)
        return ()

    lax.fori_loop(0, _NW, _writer, (), unroll=False)

    def _mx(z, mx):
        return jnp.maximum(mx, deg_v[pl.ds(z * 16, 16)])

    mx = lax.fori_loop(0, _RPT // 16, _mx, jnp.zeros((16,), jnp.int32),
                       unroll=False)
    rmax = lax.reduce_max(mx, (0,))

    lanes = lax.iota(jnp.int32, 16)
    limit = jnp.int32(_PCAP - 64)

    def _round(r, ptr):
        def _cg(g, p):
            vals = ell_v[pl.ds(r * _ESTRIDE + g * 16, 16)]
            mask = vals != _N
            incl = plsc.cumsum(mask.astype(jnp.int32))
            pos = p + incl - 1
            pos = jnp.where(mask & (pos < limit), pos, _PTRASH)
            plsc.store_scatter(plist_v, [pos], vals)
            plsc.store_scatter(llist_v, [pos], g * 16 + lanes)
            return jnp.minimum(p + incl[15], limit)

        p2 = lax.fori_loop(0, _RPT // 16, _cg, ptr, unroll=False)
        plsc.store_scatter(plist_v, [p2 + lanes], jnp.full((16,), _N,
                                                          jnp.int32))
        plsc.store_scatter(llist_v, [p2 + lanes],
                           jnp.full((16,), _DUMMY, jnp.int32))
        return (p2 + 15) & ~jnp.int32(15)

    plen = lax.fori_loop(0, jnp.minimum(rmax, _RMAX), _round,
                         jnp.int32(0), unroll=False)
    for k in range(_CHUNK // 16):
        plsc.store_scatter(plist_v, [plen + lanes + k * 16],
                           jnp.full((16,), _N, jnp.int32))
        plsc.store_scatter(llist_v, [plen + lanes + k * 16],
                           jnp.full((16,), _DUMMY, jnp.int32))
    plen = lax.div(plen + _CHUNK - 1, jnp.int32(_CHUNK)) * _CHUNK

    rv = jnp.where(lanes == 0, rmax, 0)
    rv = jnp.where(lanes == 1, plen, rv)
    rv_v[pl.ds(0, 16)] = rv
    pltpu.sync_copy(rv_v, rounds_hbm.at[w])
    pltpu.sync_copy(plist_v.at[pl.ds(0, _PCAP)], plist_hbm.at[w])
    pltpu.sync_copy(llist_v.at[pl.ds(0, _PCAP)], llist_hbm.at[w])


_ell_kernel = pl.kernel(
    _ell_body,
    out_type=[
        jax.ShapeDtypeStruct((_NW, _PCAP), jnp.int32),
        jax.ShapeDtypeStruct((_NW, _PCAP), jnp.int32),
        jax.ShapeDtypeStruct((_NW, 16), jnp.int32),
    ],
    mesh=_SC_MESH,
    scratch_types=[
        pltpu.VMEM((_EFLAT + 16,), jnp.int32),
        pltpu.VMEM((_ESTRIDE,), jnp.int32),
        pltpu.VMEM((4 * _CAP,), jnp.int32),
        pltpu.VMEM((32,), jnp.int32),
        pltpu.VMEM((16,), jnp.int32),
        pltpu.VMEM((_PTRASH + 16,), jnp.int32),
        pltpu.VMEM((_PTRASH + 16,), jnp.int32),
        pltpu.SemaphoreType.DMA,
    ],
    compiler_params=pltpu.CompilerParams(needs_layout_passes=False),
)


def _agg_body(m_hbm, s_hbm, bsrc_hbm, bloc_hbm, plist_hbm, llist_hbm,
              rounds_hbm, countsT_hbm, y_hbm,
              agg_v, rows_v, cells_v, counts_v, rv_v,
              csem, gsem):
    cid = lax.axis_index("c")
    sid = lax.axis_index("s")
    w = cid * _NS + sid
    base = w * _RPT

    pltpu.sync_copy(s_hbm.at[pl.ds(base, _RPT)], agg_v.at[pl.ds(0, _RPT)])
    pltpu.sync_copy(countsT_hbm.at[w], counts_v)
    pltpu.sync_copy(rounds_hbm.at[w], rv_v)
    rv = rv_v[pl.ds(0, 16)]
    rmax = rv[0]
    plen = rv[1]
    fast = (rmax <= _RMAX) & (plen < _PCAP - 64)

    @pl.when(fast)
    def _():
        pltpu.sync_copy(plist_hbm.at[w], cells_v.at[pl.ds(0, _PCAP)])
        pltpu.sync_copy(llist_hbm.at[w], cells_v.at[pl.ds(_PCAP, _PCAP)])
        nchunk = lax.div(plen, jnp.int32(_CHUNK))
        lanes = lax.iota(jnp.int32, 16)

        @pl.when(nchunk > 0)
        def _():
            pltpu.async_copy(m_hbm.at[cells_v.at[pl.ds(0, _CHUNK)]],
                             rows_v.at[0], gsem)

        def _chunk(c, _):
            rb = c % 2
            pltpu.make_async_copy(
                m_hbm.at[cells_v.at[pl.ds(c * _CHUNK, _CHUNK)]],
                rows_v.at[rb], gsem).wait()

            @pl.when(c + 1 < nchunk)
            def _():
                pltpu.async_copy(
                    m_hbm.at[cells_v.at[pl.ds((c + 1) * _CHUNK, _CHUNK)]],
                    rows_v.at[1 - rb], gsem)

            rbv = jnp.full((16,), rb, jnp.int32)

            def _g(gg, _):
                dv = cells_v[pl.ds(_PCAP + c * _CHUNK + gg * 16, 16)]
                row16 = gg * 16 + lanes

                def _col(c2, _):
                    c2v = jnp.full((16,), c2, jnp.int32)
                    vals = plsc.load_gather(rows_v, [rbv, row16, c2v])
                    plsc.addupdate_scatter(agg_v, [dv, c2v], vals)
                    return ()

                lax.fori_loop(0, _D, _col, (), unroll=8)
                return ()

            lax.fori_loop(0, _CHUNK // 16, _g, (), unroll=False)
            return ()

        lax.fori_loop(0, nchunk, _chunk, (), unroll=False)

    @pl.when(jnp.logical_not(fast))
    def _fallback():
        _agg_cell_walk(m_hbm, bsrc_hbm, bloc_hbm, w,
                       agg_v, rows_v, cells_v, counts_v, csem, gsem)

    pltpu.sync_copy(agg_v.at[pl.ds(0, _RPT)], y_hbm.at[pl.ds(base, _RPT)])


def _agg_cell_walk(m_hbm, bsrc_hbm, bloc_hbm, w,
                   agg_v, rows_v, cells_v, counts_v, csem, gsem):
    pltpu.async_copy(bsrc_hbm.at[w, 0], cells_v.at[pl.ds(0, _CAP)], csem)
    pltpu.async_copy(bloc_hbm.at[w, 0], cells_v.at[pl.ds(_CAP, _CAP)], csem)

    def _writer(i, _):
        coff = (i % 2) * 2 * _CAP
        pltpu.make_async_copy(bsrc_hbm.at[w, i],
                              cells_v.at[pl.ds(coff, _CAP)], csem).wait()
        pltpu.make_async_copy(bloc_hbm.at[w, i],
                              cells_v.at[pl.ds(coff + _CAP, _CAP)],
                              csem).wait()
        cnt = plsc.load_gather(counts_v, [jnp.full((16,), i, jnp.int32)])[0]
        nchunk = lax.div(cnt + _CHUNK - 1, jnp.int32(_CHUNK))

        @pl.when(i < _NW - 1)
        def _():
            noff = 2 * _CAP - coff
            pltpu.async_copy(bsrc_hbm.at[w, i + 1],
                             cells_v.at[pl.ds(noff, _CAP)], csem)
            pltpu.async_copy(bloc_hbm.at[w, i + 1],
                             cells_v.at[pl.ds(noff + _CAP, _CAP)], csem)

        @pl.when(nchunk > 0)
        def _():
            pltpu.async_copy(
                m_hbm.at[cells_v.at[pl.ds(coff, _CHUNK)]],
                rows_v.at[0], gsem)

        def _chunk(c, _):
            rb = c % 2
            pltpu.make_async_copy(
                m_hbm.at[cells_v.at[pl.ds(coff + c * _CHUNK, _CHUNK)]],
                rows_v.at[rb], gsem).wait()

            @pl.when(c + 1 < nchunk)
            def _():
                pltpu.async_copy(
                    m_hbm.at[cells_v.at[pl.ds(coff + (c + 1) * _CHUNK,
                                              _CHUNK)]],
                    rows_v.at[1 - rb], gsem)

            def _grp16(gg, _):
                dv = cells_v[pl.ds(coff + _CAP + c * _CHUNK + gg * 16, 16)]
                nq = _D // 16
                vals = [rows_v[rb, gg * 16, pl.ds(q * 16, 16)]
                        for q in range(nq)]
                for l in range(16):
                    dl = dv[l]
                    nvals = []
                    for q in range(nq):
                        plsc.addupdate(
                            agg_v.at[dl, pl.ds(q * 16, 16)], vals[q])
                        if l < 15:
                            nvals.append(
                                rows_v[rb, gg * 16 + l + 1,
                                       pl.ds(q * 16, 16)])
                    vals = nvals
                return ()

            lax.fori_loop(0, _CHUNK // 16, _grp16, (), unroll=False)
            return ()

        lax.fori_loop(0, nchunk, _chunk, (), unroll=False)
        return ()

    lax.fori_loop(0, _NW, _writer, (), unroll=False)


_agg_kernel = pl.kernel(
    _agg_body,
    out_type=jax.ShapeDtypeStruct((_NPAD, _D), jnp.float32),
    mesh=_SC_MESH,
    scratch_types=[
        pltpu.VMEM((_RPT + 1, _D), jnp.float32),
        pltpu.VMEM((2, _CHUNK, _D), jnp.float32),
        pltpu.VMEM((4 * _CAP,), jnp.int32),
        pltpu.VMEM((32,), jnp.int32),
        pltpu.VMEM((16,), jnp.int32),
        pltpu.SemaphoreType.DMA,
        pltpu.SemaphoreType.DMA,
    ],
    compiler_params=pltpu.CompilerParams(needs_layout_passes=False),
)


def _pool_body(y_ref, b_ref, wout_ref, bout_ref, o_ref, acc_ref):
    i = pl.program_id(0)
    h = jnp.maximum(y_ref[...], 0.0)
    b = b_ref[0, 0, :]
    onehot = jnp.equal(
        jnp.broadcast_to(b[:, None], (_BLK, _G)),
        lax.broadcasted_iota(jnp.int32, (_BLK, _G), 1),
    ).astype(jnp.float32)
    part = lax.dot_general(onehot, h, (((0,), (0,)), ((), ())),
                           preferred_element_type=jnp.float32)

    @pl.when(i == 0)
    def _():
        acc_ref[...] = part

    @pl.when(i > 0)
    def _():
        acc_ref[...] += part

    @pl.when(i == _NBLK - 1)
    def _():
        o_ref[...] = (jnp.dot(acc_ref[...], wout_ref[...],
                              preferred_element_type=jnp.float32)
                      + bout_ref[...])


def _pool(y, batch3d, wout, bout2d):
    return pl.pallas_call(
        _pool_body,
        grid=(_NBLK,),
        in_specs=[
            pl.BlockSpec((_BLK, _D), lambda i: (i, 0)),
            pl.BlockSpec((1, 1, _BLK), lambda i: (i, 0, 0)),
            pl.BlockSpec((_D, _T), lambda i: (0, 0)),
            pl.BlockSpec((1, _T), lambda i: (0, 0)),
        ],
        out_specs=pl.BlockSpec((_G, _T), lambda i: (0, 0)),
        out_shape=jax.ShapeDtypeStruct((_G, _T), jnp.float32),
        scratch_shapes=[pltpu.VMEM((_G, _D), jnp.float32)],
    )(y, batch3d, wout, bout2d)


def kernel(x, edge_index, batch,
           Wm0, bm0, Ws0, bs0,
           Wm1, bm1, Ws1, bs1,
           Wm2, bm2, Ws2, bs2,
           Wout, bout):
    src = edge_index[0]
    dst = edge_index[1]
    epad = _EPAD - _E
    src1d = jnp.concatenate([src, jnp.zeros((epad,), jnp.int32)])
    dst1d = jnp.concatenate([dst, jnp.full((epad,), _NPAD, jnp.int32)])
    xp = jnp.pad(x, ((0, _NPAD - _N), (0, 0)))
    batch3d = jnp.concatenate(
        [batch, jnp.full((_NPAD - _N,), _G, jnp.int32)]).reshape(
            _NBLK, 1, _BLK)
    bout2d = bout.reshape(1, _T)

    bsrc, bloc, counts = _bin_kernel(src1d, dst1d)
    countsT = counts.T.copy()
    plist, llist, rounds = _ell_kernel(bsrc, bloc, countsT)

    m, s = _dense(xp, Wm0, bm0, Ws0, bs0, apply_relu=False)
    y = _agg_kernel(m, s, bsrc, bloc, plist, llist, rounds, countsT)
    m, s = _dense(y, Wm1, bm1, Ws1, bs1, apply_relu=True)
    y = _agg_kernel(m, s, bsrc, bloc, plist, llist, rounds, countsT)
    m, s = _dense(y, Wm2, bm2, Ws2, bs2, apply_relu=True)
    y = _agg_kernel(m, s, bsrc, bloc, plist, llist, rounds, countsT)
    return _pool(y, batch3d, Wout, bout2d)


# speedup vs baseline: 4.1417x; 4.1417x over previous
import functools

import jax
import jax.numpy as jnp
from jax import lax
from jax.experimental import pallas as pl
from jax.experimental.pallas import tpu as pltpu
from jax.experimental.pallas import tpu_sc as plsc

_N = 10000
_E = 160000
_D = 256
_G = 64
_T = 100

_NC = 2
_NS = 16
_NW = _NC * _NS

_EPW = 5120
_EPAD = _EPW * _NW
_EGRP = _EPW // 16

_RPT = 320
_NPAD = _NW * _RPT
_DUMMY = _RPT

_CHUNK = 48
_CAP = 5248
_TRASH = _CAP + 48
_CBUF = _CAP + 128

_RMAX = 160
_ESTRIDE = 384
_EFLAT = (_RMAX + 1) * _ESTRIDE
_PCAP = 8064
_PTRASH = _PCAP + 32

_BLK = 1024
_NBLK = _NPAD // _BLK


_SC_MESH = plsc.VectorSubcoreMesh(core_axis_name="c", subcore_axis_name="s",
                                  num_cores=_NC, num_subcores=_NS)


def _dense_body(h_ref, wm_ref, bm_ref, ws_ref, bs_ref, m_ref, s_ref, *,
                apply_relu):
    h = h_ref[...]
    if apply_relu:
        h = jnp.maximum(h, 0.0)
    i = pl.program_id(0)
    rowmask = (lax.broadcasted_iota(jnp.int32, (_BLK, 1), 0)
               + i * _BLK) < _N
    m = (jnp.dot(h, wm_ref[...], preferred_element_type=jnp.float32)
         + bm_ref[...])
    m_ref[...] = jnp.where(rowmask, m, 0.0)
    s_ref[...] = (jnp.dot(h, ws_ref[...], preferred_element_type=jnp.float32)
                  + bs_ref[...])


def _dense(h, wm, bm, ws, bs, apply_relu):
    return pl.pallas_call(
        functools.partial(_dense_body, apply_relu=apply_relu),
        grid=(_NBLK,),
        in_specs=[
            pl.BlockSpec((_BLK, _D), lambda i: (i, 0)),
            pl.BlockSpec((_D, _D), lambda i: (0, 0)),
            pl.BlockSpec((1, _D), lambda i: (0, 0)),
            pl.BlockSpec((_D, _D), lambda i: (0, 0)),
            pl.BlockSpec((1, _D), lambda i: (0, 0)),
        ],
        out_specs=[
            pl.BlockSpec((_BLK, _D), lambda i: (i, 0)),
            pl.BlockSpec((_BLK, _D), lambda i: (i, 0)),
        ],
        out_shape=[
            jax.ShapeDtypeStruct((_NPAD, _D), jnp.float32),
            jax.ShapeDtypeStruct((_NPAD, _D), jnp.float32),
        ],
    )(h, wm, bm.reshape(1, _D), ws, bs.reshape(1, _D))


def _bin_body(src_hbm, dst_hbm, bsrc_hbm, bloc_hbm, counts_hbm,
              src_v, dst_v, cand_v, counts_v, sem):
    cid = lax.axis_index("c")
    sid = lax.axis_index("s")
    w = cid * _NS + sid

    pltpu.sync_copy(src_hbm.at[pl.ds(w * _EPW, _EPW)], src_v)
    pltpu.sync_copy(dst_hbm.at[pl.ds(w * _EPW, _EPW)], dst_v)

    lanes = lax.iota(jnp.int32, 16)

    def _owner(o, carry):
        cnt_lo, cnt_hi = carry
        lo = o * _RPT
        soff = 0
        loff = _CBUF

        def _grp(g, ptr):
            s = src_v[pl.ds(g * 16, 16)]
            d = dst_v[pl.ds(g * 16, 16)]
            mask = (d >= lo) & (d < lo + _RPT)
            incl = plsc.cumsum(mask.astype(jnp.int32))
            pos = jnp.where(mask, ptr + incl - 1, _TRASH)
            plsc.store_scatter(cand_v, [soff + pos], s)
            plsc.store_scatter(cand_v, [loff + pos], d - lo)
            return ptr + incl[15]

        cnt = lax.fori_loop(0, _EGRP, _grp, jnp.int32(0), unroll=False)

        for k in range(_CHUNK // 16):
            pos = cnt + lanes + k * 16
            plsc.store_scatter(cand_v, [soff + pos],
                               jnp.zeros((16,), jnp.int32))
            plsc.store_scatter(cand_v, [loff + pos],
                               jnp.full((16,), _DUMMY, jnp.int32))

        pltpu.sync_copy(cand_v.at[pl.ds(0, _CAP)], bsrc_hbm.at[o, w])
        pltpu.sync_copy(cand_v.at[pl.ds(_CBUF, _CAP)], bloc_hbm.at[o, w])

        onehot = lanes == (o % 16)
        cnt_lo = jnp.where(onehot & (o < 16), cnt, cnt_lo)
        cnt_hi = jnp.where(onehot & (o >= 16), cnt, cnt_hi)
        return cnt_lo, cnt_hi

    cnt_lo, cnt_hi = lax.fori_loop(
        0, _NW, _owner,
        (jnp.zeros((16,), jnp.int32), jnp.zeros((16,), jnp.int32)),
        unroll=False)

    counts_v[pl.ds(0, 16)] = cnt_lo
    counts_v[pl.ds(16, 16)] = cnt_hi
    pltpu.sync_copy(counts_v, counts_hbm.at[w])


_bin_kernel = pl.kernel(
    _bin_body,
    out_type=[
        jax.ShapeDtypeStruct((_NW, _NW, _CAP), jnp.int32),
        jax.ShapeDtypeStruct((_NW, _NW, _CAP), jnp.int32),
        jax.ShapeDtypeStruct((_NW, _NW), jnp.int32),
    ],
    mesh=_SC_MESH,
    scratch_types=[
        pltpu.VMEM((_EPW,), jnp.int32),
        pltpu.VMEM((_EPW,), jnp.int32),
        pltpu.VMEM((2 * _CBUF,), jnp.int32),
        pltpu.VMEM((32,), jnp.int32),
        pltpu.SemaphoreType.DMA,
    ],
    compiler_params=pltpu.CompilerParams(needs_layout_passes=False),
)


def _ell_body(bsrc_hbm, bloc_hbm, countsT_hbm,
              plist_hbm, llist_hbm, rounds_hbm,
              ell_v, deg_v, cells_v, counts_v, rv_v,
              plist_v, llist_v, csem):
    cid = lax.axis_index("c")
    sid = lax.axis_index("s")
    w = cid * _NS + sid

    pltpu.sync_copy(countsT_hbm.at[w], counts_v)

    def _zero_deg(z, _):
        deg_v[pl.ds(z * 16, 16)] = jnp.zeros((16,), jnp.int32)
        return ()

    lax.fori_loop(0, _ESTRIDE // 16, _zero_deg, (), unroll=False)

    def _init_ell(z, _):
        ell_v[pl.ds(z * 16, 16)] = jnp.full((16,), _N, jnp.int32)
        return ()

    lax.fori_loop(0, _EFLAT // 16, _init_ell, (), unroll=False)

    pltpu.async_copy(bsrc_hbm.at[w, 0], cells_v.at[pl.ds(0, _CAP)], csem)
    pltpu.async_copy(bloc_hbm.at[w, 0], cells_v.at[pl.ds(_CAP, _CAP)], csem)

    def _writer(i, _):
        coff = (i % 2) * 2 * _CAP
        pltpu.make_async_copy(bsrc_hbm.at[w, i],
                              cells_v.at[pl.ds(coff, _CAP)], csem).wait()
        pltpu.make_async_copy(bloc_hbm.at[w, i],
                              cells_v.at[pl.ds(coff + _CAP, _CAP)],
                              csem).wait()
        cnt = plsc.load_gather(counts_v, [jnp.full((16,), i, jnp.int32)])[0]

        @pl.when(i < _NW - 1)
        def _():
            noff = 2 * _CAP - coff
            pltpu.async_copy(bsrc_hbm.at[w, i + 1],
                             cells_v.at[pl.ds(noff, _CAP)], csem)
            pltpu.async_copy(bloc_hbm.at[w, i + 1],
                             cells_v.at[pl.ds(noff + _CAP, _CAP)], csem)

        def _grp(g, _):
            srcv = cells_v[pl.ds(coff + g * 16, 16)]
            locv = cells_v[pl.ds(coff + _CAP + g * 16, 16)]
            ordinal, last = plsc.scan_count(locv)
            degv = plsc.load_gather(deg_v, [locv])
            r = degv + ordinal - 1
            rc = jnp.minimum(r, _RMAX)
            plsc.store_scatter(ell_v, [rc * _ESTRIDE + locv], srcv)
            plsc.store_scatter(deg_v, [locv], r + 1, mask=last)
            return ()

        ngrp = lax.div(cnt + 15, jnp.int32(16))
        lax.fori_loop(0, ngrp, _grp, (), unroll=False)
        return ()

    lax.fori_loop(0, _NW, _writer, (), unroll=False)

    def _mx(z, mx):
        return jnp.maximum(mx, deg_v[pl.ds(z * 16, 16)])

    mx = lax.fori_loop(0, _RPT // 16, _mx, jnp.zeros((16,), jnp.int32),
                       unroll=False)
    rmax = lax.reduce_max(mx, (0,))

    lanes = lax.iota(jnp.int32, 16)
    limit = jnp.int32(_PCAP - 64)

    def _round(r, ptr):
        def _cg(g, p):
            vals = ell_v[pl.ds(r * _ESTRIDE + g * 16, 16)]
            mask = vals != _N
            incl = plsc.cumsum(mask.astype(jnp.int32))
            pos = p + incl - 1
            pos = jnp.where(mask & (pos < limit), pos, _PTRASH)
            plsc.store_scatter(plist_v, [pos], vals)
            plsc.store_scatter(llist_v, [pos], g * 16 + lanes)
            return jnp.minimum(p + incl[15], limit)

        p2 = lax.fori_loop(0, _RPT // 16, _cg, ptr, unroll=False)
        plsc.store_scatter(plist_v, [p2 + lanes], jnp.full((16,), _N,
                                                          jnp.int32))
        plsc.store_scatter(llist_v, [p2 + lanes],
                           jnp.full((16,), _DUMMY, jnp.int32))
        return (p2 + 15) & ~jnp.int32(15)

    plen = lax.fori_loop(0, jnp.minimum(rmax, _RMAX), _round,
                         jnp.int32(0), unroll=False)
    for k in range(_CHUNK // 16):
        plsc.store_scatter(plist_v, [plen + lanes + k * 16],
                           jnp.full((16,), _N, jnp.int32))
        plsc.store_scatter(llist_v, [plen + lanes + k * 16],
                           jnp.full((16,), _DUMMY, jnp.int32))
    plen = lax.div(plen + _CHUNK - 1, jnp.int32(_CHUNK)) * _CHUNK

    rv = jnp.where(lanes == 0, rmax, 0)
    rv = jnp.where(lanes == 1, plen, rv)
    rv_v[pl.ds(0, 16)] = rv
    pltpu.sync_copy(rv_v, rounds_hbm.at[w])
    pltpu.sync_copy(plist_v.at[pl.ds(0, _PCAP)], plist_hbm.at[w])
    pltpu.sync_copy(llist_v.at[pl.ds(0, _PCAP)], llist_hbm.at[w])


_ell_kernel = pl.kernel(
    _ell_body,
    out_type=[
        jax.ShapeDtypeStruct((_NW, _PCAP), jnp.int32),
        jax.ShapeDtypeStruct((_NW, _PCAP), jnp.int32),
        jax.ShapeDtypeStruct((_NW, 16), jnp.int32),
    ],
    mesh=_SC_MESH,
    scratch_types=[
        pltpu.VMEM((_EFLAT + 16,), jnp.int32),
        pltpu.VMEM((_ESTRIDE,), jnp.int32),
        pltpu.VMEM((4 * _CAP,), jnp.int32),
        pltpu.VMEM((32,), jnp.int32),
        pltpu.VMEM((16,), jnp.int32),
        pltpu.VMEM((_PTRASH + 16,), jnp.int32),
        pltpu.VMEM((_PTRASH + 16,), jnp.int32),
        pltpu.SemaphoreType.DMA,
    ],
    compiler_params=pltpu.CompilerParams(needs_layout_passes=False),
)


def _agg_body(m_hbm, s_hbm, bsrc_hbm, bloc_hbm, plist_hbm, llist_hbm,
              rounds_hbm, countsT_hbm, y_hbm,
              agg_v, rows_v, cells_v, counts_v, rv_v,
              csem, gsem):
    cid = lax.axis_index("c")
    sid = lax.axis_index("s")
    w = cid * _NS + sid
    base = w * _RPT

    pltpu.sync_copy(s_hbm.at[pl.ds(base, _RPT)], agg_v.at[pl.ds(0, _RPT)])
    pltpu.sync_copy(countsT_hbm.at[w], counts_v)
    pltpu.sync_copy(rounds_hbm.at[w], rv_v)
    rv = rv_v[pl.ds(0, 16)]
    rmax = rv[0]
    plen = rv[1]
    fast = (rmax <= _RMAX) & (plen < _PCAP - 64)

    @pl.when(fast)
    def _():
        pltpu.sync_copy(plist_hbm.at[w], cells_v.at[pl.ds(0, _PCAP)])
        pltpu.sync_copy(llist_hbm.at[w], cells_v.at[pl.ds(_PCAP, _PCAP)])
        nchunk = lax.div(plen, jnp.int32(_CHUNK))
        lanes = lax.iota(jnp.int32, 16)

        @pl.when(nchunk > 0)
        def _():
            pltpu.async_copy(m_hbm.at[cells_v.at[pl.ds(0, _CHUNK)]],
                             rows_v.at[0], gsem)

        def _chunk(c, _):
            rb = c % 2
            pltpu.make_async_copy(
                m_hbm.at[cells_v.at[pl.ds(c * _CHUNK, _CHUNK)]],
                rows_v.at[rb], gsem).wait()

            @pl.when(c + 1 < nchunk)
            def _():
                pltpu.async_copy(
                    m_hbm.at[cells_v.at[pl.ds((c + 1) * _CHUNK, _CHUNK)]],
                    rows_v.at[1 - rb], gsem)

            def _g(gg, _):
                dv = cells_v[pl.ds(_PCAP + c * _CHUNK + gg * 16, 16)]
                nq = _D // 16
                vals = [rows_v[rb, gg * 16, pl.ds(q * 16, 16)]
                        for q in range(nq)]
                dl = dv[0]
                for l in range(16):
                    nvals = []
                    dln = None
                    for q in range(nq):
                        plsc.addupdate(
                            agg_v.at[dl, pl.ds(q * 16, 16)], vals[q])
                        if q == 0 and l < 15:
                            dln = dv[l + 1]
                        if l < 15:
                            nvals.append(
                                rows_v[rb, gg * 16 + l + 1,
                                       pl.ds(q * 16, 16)])
                    vals = nvals
                    dl = dln
                return ()

            lax.fori_loop(0, _CHUNK // 16, _g, (), unroll=False)
            return ()

        lax.fori_loop(0, nchunk, _chunk, (), unroll=False)

    @pl.when(jnp.logical_not(fast))
    def _fallback():
        _agg_cell_walk(m_hbm, bsrc_hbm, bloc_hbm, w,
                       agg_v, rows_v, cells_v, counts_v, csem, gsem)

    pltpu.sync_copy(agg_v.at[pl.ds(0, _RPT)], y_hbm.at[pl.ds(base, _RPT)])


def _agg_cell_walk(m_hbm, bsrc_hbm, bloc_hbm, w,
                   agg_v, rows_v, cells_v, counts_v, csem, gsem):
    pltpu.async_copy(bsrc_hbm.at[w, 0], cells_v.at[pl.ds(0, _CAP)], csem)
    pltpu.async_copy(bloc_hbm.at[w, 0], cells_v.at[pl.ds(_CAP, _CAP)], csem)

    def _writer(i, _):
        coff = (i % 2) * 2 * _CAP
        pltpu.make_async_copy(bsrc_hbm.at[w, i],
                              cells_v.at[pl.ds(coff, _CAP)], csem).wait()
        pltpu.make_async_copy(bloc_hbm.at[w, i],
                              cells_v.at[pl.ds(coff + _CAP, _CAP)],
                              csem).wait()
        cnt = plsc.load_gather(counts_v, [jnp.full((16,), i, jnp.int32)])[0]
        nchunk = lax.div(cnt + _CHUNK - 1, jnp.int32(_CHUNK))

        @pl.when(i < _NW - 1)
        def _():
            noff = 2 * _CAP - coff
            pltpu.async_copy(bsrc_hbm.at[w, i + 1],
                             cells_v.at[pl.ds(noff, _CAP)], csem)
            pltpu.async_copy(bloc_hbm.at[w, i + 1],
                             cells_v.at[pl.ds(noff + _CAP, _CAP)], csem)

        @pl.when(nchunk > 0)
        def _():
            pltpu.async_copy(
                m_hbm.at[cells_v.at[pl.ds(coff, _CHUNK)]],
                rows_v.at[0], gsem)

        def _chunk(c, _):
            rb = c % 2
            pltpu.make_async_copy(
                m_hbm.at[cells_v.at[pl.ds(coff + c * _CHUNK, _CHUNK)]],
                rows_v.at[rb], gsem).wait()

            @pl.when(c + 1 < nchunk)
            def _():
                pltpu.async_copy(
                    m_hbm.at[cells_v.at[pl.ds(coff + (c + 1) * _CHUNK,
                                              _CHUNK)]],
                    rows_v.at[1 - rb], gsem)

            def _grp16(gg, _):
                dv = cells_v[pl.ds(coff + _CAP + c * _CHUNK + gg * 16, 16)]
                nq = _D // 16
                vals = [rows_v[rb, gg * 16, pl.ds(q * 16, 16)]
                        for q in range(nq)]
                for l in range(16):
                    dl = dv[l]
                    nvals = []
                    for q in range(nq):
                        plsc.addupdate(
                            agg_v.at[dl, pl.ds(q * 16, 16)], vals[q])
                        if l < 15:
                            nvals.append(
                                rows_v[rb, gg * 16 + l + 1,
                                       pl.ds(q * 16, 16)])
                    vals = nvals
                return ()

            lax.fori_loop(0, _CHUNK // 16, _grp16, (), unroll=False)
            return ()

        lax.fori_loop(0, nchunk, _chunk, (), unroll=False)
        return ()

    lax.fori_loop(0, _NW, _writer, (), unroll=False)


_agg_kernel = pl.kernel(
    _agg_body,
    out_type=jax.ShapeDtypeStruct((_NPAD, _D), jnp.float32),
    mesh=_SC_MESH,
    scratch_types=[
        pltpu.VMEM((_RPT + 1, _D), jnp.float32),
        pltpu.VMEM((2, _CHUNK, _D), jnp.float32),
        pltpu.VMEM((4 * _CAP,), jnp.int32),
        pltpu.VMEM((32,), jnp.int32),
        pltpu.VMEM((16,), jnp.int32),
        pltpu.SemaphoreType.DMA,
        pltpu.SemaphoreType.DMA,
    ],
    compiler_params=pltpu.CompilerParams(needs_layout_passes=False),
)


def _pool_body(y_ref, b_ref, wout_ref, bout_ref, o_ref, acc_ref):
    i = pl.program_id(0)
    h = jnp.maximum(y_ref[...], 0.0)
    b = b_ref[0, 0, :]
    onehot = jnp.equal(
        jnp.broadcast_to(b[:, None], (_BLK, _G)),
        lax.broadcasted_iota(jnp.int32, (_BLK, _G), 1),
    ).astype(jnp.float32)
    part = lax.dot_general(onehot, h, (((0,), (0,)), ((), ())),
                           preferred_element_type=jnp.float32)

    @pl.when(i == 0)
    def _():
        acc_ref[...] = part

    @pl.when(i > 0)
    def _():
        acc_ref[...] += part

    @pl.when(i == _NBLK - 1)
    def _():
        o_ref[...] = (jnp.dot(acc_ref[...], wout_ref[...],
                              preferred_element_type=jnp.float32)
                      + bout_ref[...])


def _pool(y, batch3d, wout, bout2d):
    return pl.pallas_call(
        _pool_body,
        grid=(_NBLK,),
        in_specs=[
            pl.BlockSpec((_BLK, _D), lambda i: (i, 0)),
            pl.BlockSpec((1, 1, _BLK), lambda i: (i, 0, 0)),
            pl.BlockSpec((_D, _T), lambda i: (0, 0)),
            pl.BlockSpec((1, _T), lambda i: (0, 0)),
        ],
        out_specs=pl.BlockSpec((_G, _T), lambda i: (0, 0)),
        out_shape=jax.ShapeDtypeStruct((_G, _T), jnp.float32),
        scratch_shapes=[pltpu.VMEM((_G, _D), jnp.float32)],
    )(y, batch3d, wout, bout2d)


def kernel(x, edge_index, batch,
           Wm0, bm0, Ws0, bs0,
           Wm1, bm1, Ws1, bs1,
           Wm2, bm2, Ws2, bs2,
           Wout, bout):
    src = edge_index[0]
    dst = edge_index[1]
    epad = _EPAD - _E
    src1d = jnp.concatenate([src, jnp.zeros((epad,), jnp.int32)])
    dst1d = jnp.concatenate([dst, jnp.full((epad,), _NPAD, jnp.int32)])
    xp = jnp.pad(x, ((0, _NPAD - _N), (0, 0)))
    batch3d = jnp.concatenate(
        [batch, jnp.full((_NPAD - _N,), _G, jnp.int32)]).reshape(
            _NBLK, 1, _BLK)
    bout2d = bout.reshape(1, _T)

    bsrc, bloc, counts = _bin_kernel(src1d, dst1d)
    countsT = counts.T.copy()
    plist, llist, rounds = _ell_kernel(bsrc, bloc, countsT)

    m, s = _dense(xp, Wm0, bm0, Ws0, bs0, apply_relu=False)
    y = _agg_kernel(m, s, bsrc, bloc, plist, llist, rounds, countsT)
    m, s = _dense(y, Wm1, bm1, Ws1, bs1, apply_relu=True)
    y = _agg_kernel(m, s, bsrc, bloc, plist, llist, rounds, countsT)
    m, s = _dense(y, Wm2, bm2, Ws2, bs2, apply_relu=True)
    y = _agg_kernel(m, s, bsrc, bloc, plist, llist, rounds, countsT)
    return _pool(y, batch3d, Wout, bout2d)
